# Initial kernel scaffold; baseline (speedup 1.0000x reference)
import jax
import jax.numpy as jnp
from jax.experimental import pallas as pl


def kernel(h, edge_index, W_src, W_dst):
    raise NotImplementedError("write your pallas kernel here")


# speedup vs baseline: 4.4468x; 4.4468x over previous
import functools

import jax
import jax.numpy as jnp
from jax import lax
from jax.experimental import pallas as pl
from jax.experimental.pallas import tpu as pltpu
from jax.experimental.pallas import tpu_sc as plsc


def _proj_body(h_ref, wsrc_ref, wdst_ref, hsrc_out, hdst_out):
    x = h_ref[...]
    dn = (((1,), (1,)), ((), ()))
    hsrc_out[...] = lax.dot_general(x, wsrc_ref[...], dn,
                                    preferred_element_type=jnp.float32)
    hdst_out[...] = lax.dot_general(x, wdst_ref[...], dn,
                                    preferred_element_type=jnp.float32)


def _project(h, W_src, W_dst):
    n, d_in = h.shape
    d_out = W_src.shape[0]
    blk = 1000
    grid = n // blk
    return pl.pallas_call(
        _proj_body,
        grid=(grid,),
        in_specs=[
            pl.BlockSpec((blk, d_in), lambda i: (i, 0)),
            pl.BlockSpec((d_out, d_in), lambda i: (0, 0)),
            pl.BlockSpec((d_out, d_in), lambda i: (0, 0)),
        ],
        out_specs=[
            pl.BlockSpec((blk, d_out), lambda i: (i, 0)),
            pl.BlockSpec((blk, d_out), lambda i: (i, 0)),
        ],
        out_shape=[
            jax.ShapeDtypeStruct((n, d_out), jnp.float32),
            jax.ShapeDtypeStruct((n, d_out), jnp.float32),
        ],
    )(h, W_src, W_dst)


def _make_edge_kernel(e_total, d, epw, chunk, nc, ns):
    nchunk = epw // chunk
    mesh = plsc.VectorSubcoreMesh(core_axis_name="c", subcore_axis_name="s")

    @functools.partial(
        pl.kernel,
        out_type=jax.ShapeDtypeStruct((e_total, d), jnp.float32),
        mesh=mesh,
        scratch_types=[
            pltpu.VMEM((epw,), jnp.int32),
            pltpu.VMEM((epw,), jnp.int32),
            pltpu.VMEM((chunk, d), jnp.float32),
            pltpu.VMEM((chunk, d), jnp.float32),
            pltpu.SemaphoreType.DMA,
            pltpu.SemaphoreType.DMA,
        ],
    )
    def edge_kernel(hsrc_hbm, hdst_hbm, src_hbm, dst_hbm, out_hbm,
                    idx_s, idx_d, rows_a, rows_b, sem_a, sem_b):
        wid = lax.axis_index("s") * nc + lax.axis_index("c")
        base = wid * epw
        pltpu.sync_copy(src_hbm.at[pl.ds(base, epw)], idx_s)
        pltpu.sync_copy(dst_hbm.at[pl.ds(base, epw)], idx_d)

        def chunk_body(j, carry):
            off = j * chunk
            cp_a = pltpu.async_copy(
                hsrc_hbm.at[idx_s.at[pl.ds(off, chunk)]], rows_a, sem_a)
            cp_b = pltpu.async_copy(
                hdst_hbm.at[idx_d.at[pl.ds(off, chunk)]], rows_b, sem_b)
            cp_a.wait()
            cp_b.wait()

            def add_row(r, c2):
                for cc in range(d // 16):
                    sl = pl.ds(cc * 16, 16)
                    plsc.addupdate(rows_a.at[r, sl], rows_b[r, sl])
                return c2

            lax.fori_loop(0, chunk, add_row, 0, unroll=False)
            pltpu.sync_copy(rows_a, out_hbm.at[pl.ds(base + off, chunk)])
            return carry

        lax.fori_loop(0, nchunk, chunk_body, 0, unroll=False)

    return edge_kernel


def kernel(h, edge_index, W_src, W_dst):
    n, d_in = h.shape
    e_total = edge_index.shape[1]
    d = W_src.shape[0]

    h_src, h_dst = _project(h, W_src, W_dst)

    info = plsc.get_sparse_core_info()
    nc, ns = info.num_cores, info.num_subcores
    nw = nc * ns
    epw = e_total // nw
    chunk = 80

    src = edge_index[0]
    dst = edge_index[1]

    edge_kernel = _make_edge_kernel(e_total, d, epw, chunk, nc, ns)
    return edge_kernel(h_src, h_dst, src, dst)
